# baseline (speedup 1.0000x reference)
import jax
import jax.numpy as jnp
from jax import lax
from jax.experimental import pallas as pl
from jax.experimental.pallas import tpu as pltpu
from jax.experimental.pallas import tpu_sc as plsc

N = 50_000
E = 1_600_000
D = 16
NC, NS = 2, 16
NPADR = 50_176
DUMMY = N
RPT = NPADR // NS
ZR = 392

EPT_A = E // NS
EPT_C = E // (NC * NS)
GROUP = 2048
CH = 128
NFULLG_A = EPT_A // GROUP
TAILG_A = EPT_A - NFULLG_A * GROUP

_MESH = plsc.VectorSubcoreMesh(core_axis_name="c", subcore_axis_name="s")
_SC_PARAMS = pltpu.CompilerParams(use_tc_tiling_on_sc=False,
                                  needs_layout_passes=False)

_IOTA = lambda: jax.lax.iota(jnp.int32, 16)


def _masked_idx(dstb, maskb, idxb, nedges):
    nfull, rem = divmod(nedges, CH)

    def row_body(j, _):
        for l in range(CH // 16):
            d = dstb[pl.ds(j * CH + l * 16, 16)]
            m = maskb[pl.ds(j * CH + l * 16, 16)]
            idxb[j, pl.ds(l * 16, 16)] = jnp.where(
                m != 0, d, jnp.full((16,), DUMMY, jnp.int32))
        return 0

    lax.fori_loop(0, nfull, row_body, 0)
    if rem:
        for l in range(CH // 16):
            if (l + 1) * 16 <= rem:
                d = dstb[pl.ds(nfull * CH + l * 16, 16)]
                m = maskb[pl.ds(nfull * CH + l * 16, 16)]
                idxb[nfull, pl.ds(l * 16, 16)] = jnp.where(
                    m != 0, d, jnp.full((16,), DUMMY, jnp.int32))
            else:
                idxb[nfull, pl.ds(l * 16, 16)] = jnp.full((16,), DUMMY,
                                                          jnp.int32)


def _scatter_body(dst_hbm, mask_hbm, et_hbm, it_hbm, agg_hbm,
                  tbuf, vals, dstb, maskb, idxb, zbuf, agg):
    c = lax.axis_index("c")
    s = lax.axis_index("s")

    def zb(i, _):
        zbuf[i] = jnp.zeros((D,), jnp.float32)
        return 0

    lax.fori_loop(0, ZR, zb, 0)
    for r in range(RPT // ZR):
        pltpu.sync_copy(zbuf, agg.at[pl.ds(s * RPT + r * ZR, ZR)])
    plsc.subcore_barrier()

    tile_base = s * EPT_A

    def process(xt_hbm, base, nedges):
        nfull, rem = divmod(nedges, CH)
        nchunks = nfull + (1 if rem else 0)
        pltpu.sync_copy(dst_hbm.at[pl.ds(base, nedges)], dstb.at[pl.ds(0, nedges)])
        pltpu.sync_copy(mask_hbm.at[pl.ds(base, nedges)], maskb.at[pl.ds(0, nedges)])
        pltpu.sync_copy(xt_hbm.at[:, pl.ds(base, nedges)],
                        tbuf.at[:, pl.ds(0, nedges)])

        def t_body(k, _):
            ridx = k * 16 + _IOTA()
            for f in range(D):
                v = tbuf[f, pl.ds(k * 16, 16)]
                plsc.store_scatter(vals, [ridx, jnp.full((16,), f, jnp.int32)], v)
            return 0

        lax.fori_loop(0, nedges // 16, t_body, 0)
        _masked_idx(dstb, maskb, idxb, nedges)
        for j in range(nchunks):
            pltpu.sync_copy(vals.at[pl.ds(j * CH, CH)], agg.at[idxb.at[j]],
                            add=True)

    def run(xt_hbm):
        def g_body(g, _):
            process(xt_hbm, tile_base + g * GROUP, GROUP)
            return 0

        lax.fori_loop(0, NFULLG_A, g_body, 0)
        process(xt_hbm, tile_base + NFULLG_A * GROUP, TAILG_A)

    @pl.when(c == 0)
    def _run_equi():
        run(et_hbm)

    @pl.when(c != 0)
    def _run_inv():
        run(it_hbm)

    plsc.subcore_barrier()
    pltpu.sync_copy(agg.at[pl.ds(s * RPT, RPT)],
                    agg_hbm.at[pl.ds(c * NPADR + s * RPT, RPT)])


_scatter_call = pl.kernel(
    _scatter_body,
    out_type=[jax.ShapeDtypeStruct((NC * NPADR, D), jnp.float32)],
    mesh=_MESH,
    scratch_types=[
        pltpu.VMEM((D, GROUP), jnp.float32),
        pltpu.VMEM((GROUP, D), jnp.float32),
        pltpu.VMEM((GROUP,), jnp.int32),
        pltpu.VMEM((GROUP,), jnp.int32),
        pltpu.VMEM((GROUP // CH, CH), jnp.int32),
        pltpu.VMEM((ZR, D), jnp.float32),
        pltpu.VMEM_SHARED((NPADR, D), jnp.float32),
    ],
    compiler_params=_SC_PARAMS,
)


CG = 1024


def _gather_body(src_hbm, ne_hbm, ni_hbm, oet_hbm, oit_hbm,
                 idxf, rows_e, rows_i, tbe, tbi, sem):
    c = lax.axis_index("c")
    s = lax.axis_index("s")
    wid = s * NC + c
    tile_base = wid * EPT_C

    def process(base, nedges):
        nfull, rem = divmod(nedges, CH)
        nchunks = nfull + (1 if rem else 0)
        pltpu.sync_copy(src_hbm.at[pl.ds(base, nedges)], idxf.at[pl.ds(0, nedges)])
        descs = []
        for j in range(nchunks):
            nrows = CH if j < nfull else rem
            isl = idxf.at[pl.ds(j * CH, nrows)]
            descs.append(pltpu.async_copy(ne_hbm.at[isl],
                                          rows_e.at[pl.ds(j * CH, nrows)], sem))
            descs.append(pltpu.async_copy(ni_hbm.at[isl],
                                          rows_i.at[pl.ds(j * CH, nrows)], sem))
        for dsc in descs:
            dsc.wait()

        def t_body(k, _):
            ridx = k * 16 + _IOTA()
            for f in range(D):
                fidx = jnp.full((16,), f, jnp.int32)
                tbe[f, pl.ds(k * 16, 16)] = plsc.load_gather(rows_e, [ridx, fidx])
                tbi[f, pl.ds(k * 16, 16)] = plsc.load_gather(rows_i, [ridx, fidx])
            return 0

        lax.fori_loop(0, nedges // 16, t_body, 0)
        pltpu.sync_copy(tbe.at[:, pl.ds(0, nedges)],
                        oet_hbm.at[:, pl.ds(base, nedges)])
        pltpu.sync_copy(tbi.at[:, pl.ds(0, nedges)],
                        oit_hbm.at[:, pl.ds(base, nedges)])

    def g_body(g, _):
        process(tile_base + g * CG, CG)
        return 0

    lax.fori_loop(0, EPT_C // CG, g_body, 0)
    if EPT_C % CG:
        process(tile_base + (EPT_C // CG) * CG, EPT_C % CG)


_gather_call = pl.kernel(
    _gather_body,
    out_type=[jax.ShapeDtypeStruct((D, E), jnp.float32),
              jax.ShapeDtypeStruct((D, E), jnp.float32)],
    mesh=_MESH,
    scratch_types=[
        pltpu.VMEM((CG,), jnp.int32),
        pltpu.VMEM((CG, D), jnp.float32),
        pltpu.VMEM((CG, D), jnp.float32),
        pltpu.VMEM((D, CG), jnp.float32),
        pltpu.VMEM((D, CG), jnp.float32),
        pltpu.SemaphoreType.DMA,
    ],
    compiler_params=_SC_PARAMS,
)


BR = 3136


def _node_body(pe, pi, wee, wei, wie, wii, wfe, wfi, bfi, ne, ni):
    ae = pe[...]
    ai = pi[...]
    f32 = jnp.float32
    he = jnp.tanh(jnp.dot(ae, wee[...], preferred_element_type=f32)
                  + jnp.dot(ai, wie[...], preferred_element_type=f32))
    hi = jnp.maximum(jnp.dot(ai, wii[...], preferred_element_type=f32)
                     + jnp.dot(ae, wei[...], preferred_element_type=f32), 0.0)
    oe = (jnp.dot(he, wfe[0:D, :], preferred_element_type=f32)
          + jnp.dot(hi, wfe[D:2 * D, :], preferred_element_type=f32) + he)
    oi = (jnp.dot(jnp.abs(he), wfi[0:D, :], preferred_element_type=f32)
          + jnp.dot(hi, wfi[D:2 * D, :], preferred_element_type=f32)
          + bfi[...] + hi)
    ne[...] = oe
    ni[...] = oi


def _w_spec():
    return pl.BlockSpec((D, D), lambda i: (0, 0))


_node_call = pl.pallas_call(
    _node_body,
    grid=(NPADR // BR,),
    in_specs=[
        pl.BlockSpec((BR, D), lambda i: (i, 0)),
        pl.BlockSpec((BR, D), lambda i: (i + NPADR // BR, 0)),
        _w_spec(), _w_spec(), _w_spec(), _w_spec(),
        pl.BlockSpec((2 * D, D), lambda i: (0, 0)),
        pl.BlockSpec((2 * D, D), lambda i: (0, 0)),
        pl.BlockSpec((1, D), lambda i: (0, 0)),
    ],
    out_specs=[pl.BlockSpec((BR, D), lambda i: (i, 0)),
               pl.BlockSpec((BR, D), lambda i: (i, 0))],
    out_shape=[jax.ShapeDtypeStruct((NPADR, D), jnp.float32),
               jax.ShapeDtypeStruct((NPADR, D), jnp.float32)],
)


@jax.jit
def kernel(edge_index, equi_edge_attr, inv_edge_attr, undirected_mask,
           W_ee, W_ei, W_ie, W_ii, W_fe, W_fi, b_fi):
    src = edge_index[0]
    dst = edge_index[1]
    mask_i = undirected_mask.astype(jnp.int32)
    (agg,) = _scatter_call(dst, mask_i, equi_edge_attr.T, inv_edge_attr.T)
    node_e, node_i = _node_call(agg, agg, W_ee, W_ei, W_ie, W_ii,
                                W_fe, W_fi, b_fi.reshape(1, D))
    oet, oit = _gather_call(src, node_e, node_i)
    return oet.T, oit.T


# speedup vs baseline: 3.6816x; 3.6816x over previous
import jax
import jax.numpy as jnp
from jax import lax
from jax.experimental import pallas as pl
from jax.experimental.pallas import tpu as pltpu
from jax.experimental.pallas import tpu_sc as plsc

N = 50_000
E = 1_600_000
D = 16
NC, NS = 2, 16
NPADR = 50_176
DUMMY = N
RPT = NPADR // NS
ZR = 392
TB = E // 128
CH = 128

TPT_A_LO = TB // NS
A_EXTRA = TB - TPT_A_LO * NS
GA = 8

TPT_C_LO = TB // (NC * NS)
C_EXTRA = TB - TPT_C_LO * NC * NS
GC = 4

_MESH = plsc.VectorSubcoreMesh(core_axis_name="c", subcore_axis_name="s")
_SC_PARAMS = pltpu.CompilerParams(use_tc_tiling_on_sc=False,
                                  needs_layout_passes=False)

_IOTA = lambda: jax.lax.iota(jnp.int32, 16)


def _scatter_body(dst_hbm, mask_hbm, qe_hbm, qi_hbm, agg_hbm,
                  tiles, vals, dstb, maskb, idxb, zbuf, agg):
    c = lax.axis_index("c")
    s = lax.axis_index("s")

    def zb(i, _):
        zbuf[i] = jnp.zeros((D,), jnp.float32)
        return 0

    lax.fori_loop(0, ZR, zb, 0)
    for r in range(RPT // ZR):
        pltpu.sync_copy(zbuf, agg.at[pl.ds(s * RPT + r * ZR, ZR)])
    plsc.subcore_barrier()

    base_t = s * TPT_A_LO + jnp.minimum(s, A_EXTRA)
    ntiles = TPT_A_LO + jnp.where(s < A_EXTRA, 1, 0)

    A4 = _IOTA() // 8
    F4 = _IOTA() % 8

    def process(q_hbm, bt, nt):
        ne = nt * 128
        pltpu.sync_copy(dst_hbm.at[pl.ds(bt * 128, ne)], dstb.at[pl.ds(0, ne)])
        pltpu.sync_copy(mask_hbm.at[pl.ds(bt * 128, ne)], maskb.at[pl.ds(0, ne)])
        pltpu.sync_copy(q_hbm.at[:, pl.ds(bt, nt)], tiles.at[:, pl.ds(0, nt)])

        def t_body(k, _):
            t_loc = jnp.zeros((16,), jnp.int32) + k // 8
            lane0 = (k % 8) * 16
            for i in range(16):
                v = plsc.load_gather(
                    tiles, [A4, t_loc, F4,
                            jnp.zeros((16,), jnp.int32) + (lane0 + i)])
                vals[k * 16 + i] = v
            return 0

        lax.fori_loop(0, ne // 16, t_body, 0)

        def m_body(j, _):
            for l in range(CH // 16):
                d = dstb[pl.ds(j * CH + l * 16, 16)]
                m = maskb[pl.ds(j * CH + l * 16, 16)]
                idxb[j, pl.ds(l * 16, 16)] = jnp.where(
                    m != 0, d, jnp.full((16,), DUMMY, jnp.int32))
            return 0

        lax.fori_loop(0, nt, m_body, 0)
        for j in range(nt):
            pltpu.sync_copy(vals.at[pl.ds(j * CH, CH)], agg.at[idxb.at[j]],
                            add=True)

    def run(q_hbm):
        nfull = ntiles // GA

        def g_body(g, _):
            process(q_hbm, base_t + g * GA, GA)
            return 0

        lax.fori_loop(0, nfull, g_body, 0)

        def tail_body(g, _):
            process(q_hbm, base_t + nfull * GA + g, 1)
            return 0

        lax.fori_loop(0, ntiles - nfull * GA, tail_body, 0)

    @pl.when(c == 0)
    def _run_equi():
        run(qe_hbm)

    @pl.when(c != 0)
    def _run_inv():
        run(qi_hbm)

    plsc.subcore_barrier()
    pltpu.sync_copy(agg.at[pl.ds(s * RPT, RPT)],
                    agg_hbm.at[pl.ds(c * NPADR + s * RPT, RPT)])


_scatter_call = pl.kernel(
    _scatter_body,
    out_type=[jax.ShapeDtypeStruct((NC * NPADR, D), jnp.float32)],
    mesh=_MESH,
    scratch_types=[
        pltpu.VMEM((2, GA, 8, 128), jnp.float32),
        pltpu.VMEM((GA * 128, D), jnp.float32),
        pltpu.VMEM((GA * 128,), jnp.int32),
        pltpu.VMEM((GA * 128,), jnp.int32),
        pltpu.VMEM((GA, CH), jnp.int32),
        pltpu.VMEM((ZR, D), jnp.float32),
        pltpu.VMEM_SHARED((NPADR, D), jnp.float32),
    ],
    compiler_params=_SC_PARAMS,
)


def _gather_body(src_hbm, ne_hbm, ni_hbm, oe_hbm, oi_hbm,
                 idxf, rows_e, rows_i, tbe, tbi, sem):
    c = lax.axis_index("c")
    s = lax.axis_index("s")
    wid = s * NC + c
    base_t = wid * TPT_C_LO + jnp.minimum(wid, C_EXTRA)
    ntiles = TPT_C_LO + jnp.where(wid < C_EXTRA, 1, 0)

    A4 = _IOTA() // 8
    F4 = _IOTA() % 8

    def process(bt, nt):
        ne = nt * 128
        pltpu.sync_copy(src_hbm.at[pl.ds(bt * 128, ne)], idxf.at[pl.ds(0, ne)])
        descs = []
        for j in range(nt):
            isl = idxf.at[pl.ds(j * CH, CH)]
            descs.append(pltpu.async_copy(ne_hbm.at[isl],
                                          rows_e.at[pl.ds(j * CH, CH)], sem))
            descs.append(pltpu.async_copy(ni_hbm.at[isl],
                                          rows_i.at[pl.ds(j * CH, CH)], sem))
        for dsc in descs:
            dsc.wait()

        def t_body(k, _):
            t_loc = jnp.zeros((16,), jnp.int32) + k // 8
            lane0 = (k % 8) * 16
            for i in range(16):
                lv = jnp.zeros((16,), jnp.int32) + (lane0 + i)
                plsc.store_scatter(tbe, [A4, t_loc, F4, lv], rows_e[k * 16 + i])
                plsc.store_scatter(tbi, [A4, t_loc, F4, lv], rows_i[k * 16 + i])
            return 0

        lax.fori_loop(0, ne // 16, t_body, 0)
        pltpu.sync_copy(tbe.at[:, pl.ds(0, nt)], oe_hbm.at[:, pl.ds(bt, nt)])
        pltpu.sync_copy(tbi.at[:, pl.ds(0, nt)], oi_hbm.at[:, pl.ds(bt, nt)])

    nfull = ntiles // GC

    def g_body(g, _):
        process(base_t + g * GC, GC)
        return 0

    lax.fori_loop(0, nfull, g_body, 0)

    def tail_body(g, _):
        process(base_t + nfull * GC + g, 1)
        return 0

    lax.fori_loop(0, ntiles - nfull * GC, tail_body, 0)


_gather_call = pl.kernel(
    _gather_body,
    out_type=[jax.ShapeDtypeStruct((2, TB, 8, 128), jnp.float32),
              jax.ShapeDtypeStruct((2, TB, 8, 128), jnp.float32)],
    mesh=_MESH,
    scratch_types=[
        pltpu.VMEM((GC * 128,), jnp.int32),
        pltpu.VMEM((GC * 128, D), jnp.float32),
        pltpu.VMEM((GC * 128, D), jnp.float32),
        pltpu.VMEM((2, GC, 8, 128), jnp.float32),
        pltpu.VMEM((2, GC, 8, 128), jnp.float32),
        pltpu.SemaphoreType.DMA,
    ],
    compiler_params=_SC_PARAMS,
)


BR = 3136


def _node_body(pe, pi, wee, wei, wie, wii, wfe, wfi, bfi, ne, ni):
    ae = pe[...]
    ai = pi[...]
    f32 = jnp.float32
    he = jnp.tanh(jnp.dot(ae, wee[...], preferred_element_type=f32)
                  + jnp.dot(ai, wie[...], preferred_element_type=f32))
    hi = jnp.maximum(jnp.dot(ai, wii[...], preferred_element_type=f32)
                     + jnp.dot(ae, wei[...], preferred_element_type=f32), 0.0)
    oe = (jnp.dot(he, wfe[0:D, :], preferred_element_type=f32)
          + jnp.dot(hi, wfe[D:2 * D, :], preferred_element_type=f32) + he)
    oi = (jnp.dot(jnp.abs(he), wfi[0:D, :], preferred_element_type=f32)
          + jnp.dot(hi, wfi[D:2 * D, :], preferred_element_type=f32)
          + bfi[...] + hi)
    ne[...] = oe
    ni[...] = oi


def _w_spec():
    return pl.BlockSpec((D, D), lambda i: (0, 0))


_node_call = pl.pallas_call(
    _node_body,
    grid=(NPADR // BR,),
    in_specs=[
        pl.BlockSpec((BR, D), lambda i: (i, 0)),
        pl.BlockSpec((BR, D), lambda i: (i + NPADR // BR, 0)),
        _w_spec(), _w_spec(), _w_spec(), _w_spec(),
        pl.BlockSpec((2 * D, D), lambda i: (0, 0)),
        pl.BlockSpec((2 * D, D), lambda i: (0, 0)),
        pl.BlockSpec((1, D), lambda i: (0, 0)),
    ],
    out_specs=[pl.BlockSpec((BR, D), lambda i: (i, 0)),
               pl.BlockSpec((BR, D), lambda i: (i, 0))],
    out_shape=[jax.ShapeDtypeStruct((NPADR, D), jnp.float32),
               jax.ShapeDtypeStruct((NPADR, D), jnp.float32)],
)


def _tile_view(x):
    return x.T.reshape(2, 8, TB, 128).transpose(0, 2, 1, 3)


def _untile_view(q):
    return q.transpose(0, 2, 1, 3).reshape(D, E).T


@jax.jit
def kernel(edge_index, equi_edge_attr, inv_edge_attr, undirected_mask,
           W_ee, W_ei, W_ie, W_ii, W_fe, W_fi, b_fi):
    src = edge_index[0]
    dst = edge_index[1]
    mask_i = undirected_mask.astype(jnp.int32)
    (agg,) = _scatter_call(dst, mask_i, _tile_view(equi_edge_attr),
                           _tile_view(inv_edge_attr))
    node_e, node_i = _node_call(agg, agg, W_ee, W_ei, W_ie, W_ii,
                                W_fe, W_fi, b_fi.reshape(1, D))
    qoe, qoi = _gather_call(src, node_e, node_i)
    return _untile_view(qoe), _untile_view(qoi)
